# Initial kernel scaffold; baseline (speedup 1.0000x reference)
import jax
import jax.numpy as jnp
from jax.experimental import pallas as pl


def kernel(node_attr, edge_input, edge_index, conv_w, conv_b, W1, b1, W2, b2, W3, b3):
    raise NotImplementedError("write your pallas kernel here")


# speedup vs baseline: 2.1195x; 2.1195x over previous
import functools

import jax
import jax.numpy as jnp
from jax import lax
from jax.experimental import pallas as pl
from jax.experimental.pallas import tpu as pltpu
from jax.experimental.pallas import tpu_sc as plsc

H = 128


def _emb_body(w_ref, b_ref, attr_ref, out_ref):
    a = attr_ref[...]
    out_ref[...] = (
        a[:, 0 * H:1 * H] * w_ref[0]
        + a[:, 1 * H:2 * H] * w_ref[1]
        + a[:, 2 * H:3 * H] * w_ref[2]
        + a[:, 3 * H:4 * H] * w_ref[3]
        + b_ref[0]
    )


def _node_emb(node_attr2d, conv_w, conv_b):
    n = node_attr2d.shape[0]
    nb = 1000
    grid = (n // nb,)
    return pl.pallas_call(
        _emb_body,
        grid=grid,
        in_specs=[
            pl.BlockSpec(memory_space=pltpu.SMEM),
            pl.BlockSpec(memory_space=pltpu.SMEM),
            pl.BlockSpec((nb, 4 * H), lambda i: (i, 0)),
        ],
        out_specs=pl.BlockSpec((nb, H), lambda i: (i, 0)),
        out_shape=jax.ShapeDtypeStruct((n, H), jnp.float32),
    )(conv_w, conv_b, node_attr2d)


def _gather_src_tgt(node_emb, row, col):
    e = row.shape[0]
    nw = 32
    ch = 128
    n_chunks = e // ch
    base_trips = n_chunks // nw
    extra = n_chunks - base_trips * nw

    mesh = plsc.VectorSubcoreMesh(core_axis_name="c", subcore_axis_name="s")

    @functools.partial(
        pl.kernel,
        mesh=mesh,
        out_type=(
            jax.ShapeDtypeStruct((e, H), jnp.float32),
            jax.ShapeDtypeStruct((e, H), jnp.float32),
        ),
        scratch_types=[
            pltpu.VMEM((ch,), jnp.int32),
            pltpu.VMEM((ch, H), jnp.float32),
            pltpu.SemaphoreType.DMA,
        ],
    )
    def k(emb_hbm, row_hbm, col_hbm, src_out, tgt_out, idx_v, rows_v, sem):
        wid = lax.axis_index("s") * 2 + lax.axis_index("c")
        trips = base_trips + jnp.where(wid < extra, 1, 0)

        def one(idx_hbm, out_hbm, off):
            pltpu.sync_copy(idx_hbm.at[pl.ds(off, ch)], idx_v)
            pltpu.async_copy(emb_hbm.at[idx_v], rows_v, sem).wait()
            pltpu.sync_copy(rows_v, out_hbm.at[pl.ds(off, ch)])

        def body(i, carry):
            c = wid + nw * i
            off = c * ch
            one(row_hbm, src_out, off)
            one(col_hbm, tgt_out, off)
            return carry

        lax.fori_loop(0, trips, body, 0)

    return k(node_emb, row, col)


def _mlp_body(src_ref, tgt_ref, edge_ref, w1s_ref, w1t_ref, w1e_ref,
              b1_ref, w2_ref, b2_ref, w3_ref, b3_ref, out_ref):
    f32 = jnp.float32
    h = (
        jnp.dot(src_ref[...], w1s_ref[...], preferred_element_type=f32)
        + jnp.dot(tgt_ref[...], w1t_ref[...], preferred_element_type=f32)
        + jnp.dot(edge_ref[...], w1e_ref[...], preferred_element_type=f32)
        + b1_ref[...]
    )
    h = jnp.maximum(h, 0.0)
    h = jnp.maximum(
        jnp.dot(h, w2_ref[...], preferred_element_type=f32) + b2_ref[...], 0.0)
    out_ref[...] = (
        jnp.dot(h, w3_ref[...], preferred_element_type=f32) + b3_ref[...])


def _edge_mlp(src, tgt, edge_input, w1t, b1, w2t, b2, w3t, b3):
    e = src.shape[0]
    eb = 2000
    grid = (e // eb,)
    d1 = w1t.shape[1]
    d2 = w2t.shape[1]
    d3 = w3t.shape[1]
    blk = lambda r, c: pl.BlockSpec((r, c), lambda i: (i, 0))
    full = lambda r, c: pl.BlockSpec((r, c), lambda i: (0, 0))
    return pl.pallas_call(
        _mlp_body,
        grid=grid,
        in_specs=[
            blk(eb, H), blk(eb, H), blk(eb, H),
            full(H, d1), full(H, d1), full(H, d1), full(1, d1),
            full(d1, d2), full(1, d2),
            full(d2, d3), full(1, d3),
        ],
        out_specs=pl.BlockSpec((eb, d3), lambda i: (i, 0)),
        out_shape=jax.ShapeDtypeStruct((e, d3), jnp.float32),
    )(src, tgt, edge_input,
      w1t[0 * H:1 * H], w1t[1 * H:2 * H], w1t[2 * H:3 * H], b1[None, :],
      w2t, b2[None, :], w3t, b3[None, :])


def kernel(node_attr, edge_input, edge_index, conv_w, conv_b,
           W1, b1, W2, b2, W3, b3):
    n = node_attr.shape[0]
    node_attr2d = node_attr.reshape(n, 4 * H)
    emb = _node_emb(node_attr2d, conv_w, conv_b)
    row = edge_index[0].astype(jnp.int32)
    col = edge_index[1].astype(jnp.int32)
    src, tgt = _gather_src_tgt(emb, row, col)
    return _edge_mlp(src, tgt, edge_input, W1.T, b1, W2.T, b2, W3.T, b3)
